# Initial kernel scaffold; baseline (speedup 1.0000x reference)
import functools

import jax
import jax.numpy as jnp
from jax import lax
from jax.experimental import pallas as pl
from jax.experimental.pallas import tpu as pltpu
from jax.experimental.pallas import tpu_sc as plsc

D_MODEL = 64
GROUP = 128


@jax.jit
def _gather_rows(idx, W):
    n = idx.shape[0]
    info = plsc.get_sparse_core_info()
    nw = info.num_cores * info.num_subcores
    rows_per_w = n // nw
    ngroups = rows_per_w // GROUP

    mesh = plsc.VectorSubcoreMesh(core_axis_name="c", subcore_axis_name="s")

    @functools.partial(
        pl.kernel,
        mesh=mesh,
        out_type=jax.ShapeDtypeStruct((n, D_MODEL), jnp.float32),
        scratch_types=[
            pltpu.VMEM((1, GROUP), jnp.int32),
            pltpu.VMEM((GROUP, D_MODEL), jnp.float32),
            pltpu.SemaphoreType.DMA,
        ],
    )
    def k(idx_hbm, table_hbm, out_hbm, idx_v, rows_v, sem):
        wid = lax.axis_index("s") * info.num_cores + lax.axis_index("c")
        base = wid * rows_per_w

        def body(g, carry):
            start = base + g * GROUP
            pltpu.sync_copy(idx_hbm.at[pl.ds(start, GROUP)], idx_v.at[0])
            pltpu.async_copy(table_hbm.at[idx_v.at[0]], rows_v, sem).wait()
            pltpu.sync_copy(rows_v, out_hbm.at[pl.ds(start, GROUP)])
            return carry

        lax.fori_loop(0, ngroups, body, 0)

    return k(idx, W)


def kernel(dist, W):
    b, s = dist.shape
    flat = dist.reshape(b * s)
    out = _gather_rows(flat, W)
    return out.reshape(b, s, D_MODEL)


# speedup vs baseline: 3.1740x; 3.1740x over previous
import functools

import jax
import jax.numpy as jnp
from jax import lax
from jax.experimental import pallas as pl
from jax.experimental.pallas import tpu as pltpu
from jax.experimental.pallas import tpu_sc as plsc

D_MODEL = 64
GROUP = 128


@jax.jit
def _gather_rows(idx, W):
    n = idx.shape[0]
    info = plsc.get_sparse_core_info()
    nw = info.num_cores * info.num_subcores
    rows_per_w = n // nw
    ngroups = rows_per_w // GROUP

    mesh = plsc.VectorSubcoreMesh(core_axis_name="c", subcore_axis_name="s")

    @functools.partial(
        pl.kernel,
        mesh=mesh,
        out_type=jax.ShapeDtypeStruct((n, D_MODEL), jnp.float32),
        scratch_types=[
            pltpu.VMEM((1, GROUP), jnp.int32),
            pltpu.VMEM((GROUP, D_MODEL), jnp.float32),
            pltpu.SemaphoreType.DMA,
        ],
        compiler_params=pltpu.CompilerParams(use_tc_tiling_on_sc=False),
    )
    def k(idx_hbm, table_hbm, out_hbm, idx_v, rows_v, sem):
        wid = lax.axis_index("s") * info.num_cores + lax.axis_index("c")
        base = wid * rows_per_w

        def body(g, carry):
            start = base + g * GROUP
            pltpu.sync_copy(idx_hbm.at[pl.ds(start, GROUP)], idx_v.at[0])
            pltpu.async_copy(table_hbm.at[idx_v.at[0]], rows_v, sem).wait()
            pltpu.sync_copy(rows_v, out_hbm.at[pl.ds(start, GROUP)])
            return carry

        lax.fori_loop(0, ngroups, body, 0)

    return k(idx, W)


def kernel(dist, W):
    b, s = dist.shape
    flat = dist.reshape(b * s)
    out = _gather_rows(flat, W)
    return out.reshape(b, s, D_MODEL)


# speedup vs baseline: 4.2598x; 1.3421x over previous
import functools

import jax
import jax.numpy as jnp
from jax import lax
from jax.experimental import pallas as pl
from jax.experimental.pallas import tpu as pltpu
from jax.experimental.pallas import tpu_sc as plsc

D_MODEL = 64
GROUP = 128
K = 4
CHUNK = K * GROUP


@jax.jit
def _gather_rows(idx3, W):
    nw, ngroups, _ = idx3.shape
    rows_per_w = ngroups * GROUP
    n = nw * rows_per_w
    nchunks = ngroups // K
    info = plsc.get_sparse_core_info()

    mesh = plsc.VectorSubcoreMesh(core_axis_name="c", subcore_axis_name="s")

    @functools.partial(
        pl.kernel,
        mesh=mesh,
        out_type=jax.ShapeDtypeStruct((n, D_MODEL), jnp.float32),
        scratch_types=[
            pltpu.VMEM((ngroups, GROUP), jnp.int32),
            pltpu.VMEM((CHUNK, D_MODEL), jnp.float32),
            pltpu.VMEM((CHUNK, D_MODEL), jnp.float32),
            pltpu.SemaphoreType.DMA,
            pltpu.SemaphoreType.DMA,
            pltpu.SemaphoreType.DMA,
            pltpu.SemaphoreType.DMA,
        ],
        compiler_params=pltpu.CompilerParams(use_tc_tiling_on_sc=False),
    )
    def k(idx_hbm, table_hbm, out_hbm, idx_v, buf0, buf1, g0, g1, s0, s1):
        wid = lax.axis_index("s") * info.num_cores + lax.axis_index("c")
        base = wid * rows_per_w
        bufs = (buf0, buf1)
        gsems = (g0, g1)
        ssems = (s0, s1)

        def fire_gather(c, p):
            for j in range(K):
                pltpu.async_copy(
                    table_hbm.at[idx_v.at[c * K + j]],
                    bufs[p].at[pl.ds(j * GROUP, GROUP)],
                    gsems[p],
                )

        def drain_gather(c, p):
            for j in range(K):
                pltpu.make_async_copy(
                    table_hbm.at[idx_v.at[c * K + j]],
                    bufs[p].at[pl.ds(j * GROUP, GROUP)],
                    gsems[p],
                ).wait()

        def fire_store(c, p):
            pltpu.async_copy(
                bufs[p], out_hbm.at[pl.ds(base + c * CHUNK, CHUNK)], ssems[p]
            )

        def drain_store(c, p):
            pltpu.make_async_copy(
                bufs[p], out_hbm.at[pl.ds(base + c * CHUNK, CHUNK)], ssems[p]
            ).wait()

        pltpu.sync_copy(idx_hbm.at[wid], idx_v)

        fire_gather(0, 0)
        fire_gather(1, 1)
        drain_gather(0, 0)
        fire_store(0, 0)

        @pl.loop(0, (nchunks - 2) // 2)
        def _body(cc):
            for par in range(2):
                c = 1 + cc * 2 + par
                cur = (1 + par) % 2
                oth = 1 - cur
                drain_store(c - 1, oth)
                fire_gather(c + 1, oth)
                drain_gather(c, cur)
                fire_store(c, cur)

        c_last = nchunks - 1
        drain_store(c_last - 1, 0)
        drain_gather(c_last, 1)
        fire_store(c_last, 1)
        drain_store(c_last, 1)

    return k(idx3, W)


def kernel(dist, W):
    b, s = dist.shape
    n = b * s
    info = plsc.get_sparse_core_info()
    nw = info.num_cores * info.num_subcores
    idx3 = dist.reshape(nw, (n // nw) // GROUP, GROUP)
    out = _gather_rows(idx3, W)
    return out.reshape(b, s, D_MODEL)


# speedup vs baseline: 13.0624x; 3.0664x over previous
import functools

import jax
import jax.numpy as jnp
from jax import lax
from jax.experimental import pallas as pl
from jax.experimental.pallas import tpu as pltpu
from jax.experimental.pallas import tpu_sc as plsc

D_MODEL = 64
BBLK = 128
LANES = 16


@jax.jit
def _gather_t(idx4, W):
    nst, nw, _, _ = idx4.shape
    seq = nst * 8
    nbt = nw
    info = plsc.get_sparse_core_info()

    mesh = plsc.VectorSubcoreMesh(core_axis_name="c", subcore_axis_name="s")

    @functools.partial(
        pl.kernel,
        mesh=mesh,
        out_type=jax.ShapeDtypeStruct(
            (seq, D_MODEL // 8, nbt, 8, BBLK), jnp.float32
        ),
        scratch_types=[
            pltpu.VMEM((nst, 8, BBLK), jnp.int32),
        ] + [pltpu.VMEM((BBLK, D_MODEL), jnp.float32)] * 4
          + [pltpu.VMEM((D_MODEL // 8, 8, BBLK + 1), jnp.float32)] * 4
          + [pltpu.SemaphoreType.DMA] * 8,
        compiler_params=pltpu.CompilerParams(
            use_tc_tiling_on_sc=False, needs_layout_passes=False
        ),
    )
    def k(idx_hbm, table_hbm, q_hbm, idx_v,
          G0, G1, G2, G3, T0, T1, T2, T3,
          g0, g1, g2, g3, s0, s1, s2, s3):
        w = lax.axis_index("s") * info.num_cores + lax.axis_index("c")
        Gs = (G0, G1, G2, G3)
        Ts = (T0, T1, T2, T3)
        gsems = (g0, g1, g2, g3)
        ssems = (s0, s1, s2, s3)
        lane = jnp.arange(LANES, dtype=jnp.int32)
        dtv = [(lane + d16 * LANES) // 8 for d16 in range(D_MODEL // LANES)]
        drv = [(lane + d16 * LANES) % 8 for d16 in range(D_MODEL // LANES)]

        def fire_gather(st, sr, p):
            pltpu.async_copy(table_hbm.at[idx_v.at[st, sr]], Gs[p], gsems[p])

        def drain_gather(st, sr, p):
            pltpu.make_async_copy(
                table_hbm.at[idx_v.at[st, sr]], Gs[p], gsems[p]
            ).wait()

        def fire_store(s, p):
            pltpu.async_copy(
                Ts[p].at[:, :, pl.ds(0, BBLK)], q_hbm.at[s, :, w], ssems[p]
            )

        def drain_store(s, p):
            pltpu.make_async_copy(
                Ts[p].at[:, :, pl.ds(0, BBLK)], q_hbm.at[s, :, w], ssems[p]
            ).wait()

        def transpose(p):
            G = Gs[p]
            T = Ts[p]

            @plsc.parallel_loop(0, BBLK, unroll=8)
            def _b(b):
                bs = jnp.full((LANES,), b, jnp.int32)
                for d16 in range(D_MODEL // LANES):
                    v = G[b, pl.ds(d16 * LANES, LANES)]
                    plsc.store_scatter(T, [dtv[d16], drv[d16], bs], v)

        pltpu.sync_copy(idx_hbm.at[:, w], idx_v)

        def leg(st, sr, s, fire2=True, drain_prev=True):
            p = sr % 4
            if fire2:
                st2, sr2 = (st, sr + 2) if sr < 6 else (st + 1, sr - 6)
                fire_gather(st2, sr2, (sr + 2) % 4)
            drain_gather(st, sr, p)
            if drain_prev:
                drain_store(s - 4, p)
            transpose(p)
            fire_store(s, p)

        fire_gather(0, 0, 0)
        fire_gather(0, 1, 1)
        for sr in range(8):
            leg(0, sr, sr, drain_prev=(sr >= 4))

        @pl.loop(1, nst - 1)
        def _body(st):
            s = st * 8
            for sr in range(8):
                leg(st, sr, s + sr)

        st_l = nst - 1
        for sr in range(8):
            leg(st_l, sr, st_l * 8 + sr, fire2=(sr < 6))
        for s in range(seq - 4, seq):
            drain_store(s, s % 4)

    return k(idx4, W)


def kernel(dist, W):
    b, s = dist.shape
    nw = 32
    idx4 = dist.reshape(nw, BBLK, s // 8, 8).transpose(2, 0, 3, 1)
    Q = _gather_t(idx4, W)
    return Q.transpose(2, 4, 0, 1, 3).reshape(b, s, D_MODEL)
